# baseline (device time: 68450 ns/iter reference)
import jax
import jax.numpy as jnp
from jax import lax
from jax.experimental import pallas as pl
from jax.experimental.pallas import tpu as pltpu

N_DEV = 8
B, SQ, D = 2, 128, 512
HQ_PER = 8
DH = 64
BSQ = B * SQ


def kernel(x, Wq, Wo, Wk, Wv):
    def body(x_ref, wq_ref, wo_ref, wk_ref, wv_ref, out_ref,
             comm_ref, send_sems, recv_sems):
        my = lax.axis_index("i")
        left = lax.rem(my - 1 + N_DEV, N_DEV)
        right = lax.rem(my + 1, N_DEV)

        x2 = x_ref[...].reshape(BSQ, D).astype(jnp.bfloat16)
        wq = wq_ref[...].astype(jnp.bfloat16)
        q = jnp.dot(x2, wq, preferred_element_type=jnp.float32)
        q = q.astype(jnp.bfloat16)

        kv_start = my * (2 * DH)
        wk = wk_ref[:, pl.ds(kv_start, 2 * DH)].astype(jnp.bfloat16)
        wv = wv_ref[:, pl.ds(kv_start, 2 * DH)].astype(jnp.bfloat16)
        k = jnp.dot(x2, wk, preferred_element_type=jnp.float32).astype(jnp.bfloat16)
        v = jnp.dot(x2, wv, preferred_element_type=jnp.float32).astype(jnp.bfloat16)

        ctx_rows = []
        for b in range(B):
            r0 = b * SQ
            head_ctx = []
            for h in range(HQ_PER):
                kvh = h // 4
                qh = q[r0:r0 + SQ, h * DH:(h + 1) * DH]
                kh = k[r0:r0 + SQ, kvh * DH:(kvh + 1) * DH]
                vh = v[r0:r0 + SQ, kvh * DH:(kvh + 1) * DH]
                s = jnp.dot(qh, kh.T, preferred_element_type=jnp.float32) * 0.125
                m = jnp.max(s, axis=-1, keepdims=True)
                p = jnp.exp(s - m)
                l = jnp.sum(p, axis=-1, keepdims=True)
                o = jnp.dot(p.astype(jnp.bfloat16), vh,
                            preferred_element_type=jnp.float32) / l
                head_ctx.append(o.astype(jnp.bfloat16))
            ctx_rows.append(jnp.concatenate(head_ctx, axis=1))
        ctx = jnp.concatenate(ctx_rows, axis=0)

        partial = jnp.dot(ctx, wo_ref[...].astype(jnp.bfloat16),
                          preferred_element_type=jnp.float32)

        barrier_sem = pltpu.get_barrier_semaphore()
        for nbr in [left, right]:
            pl.semaphore_signal(barrier_sem, inc=1, device_id=(nbr,),
                                device_id_type=pl.DeviceIdType.MESH)
        pl.semaphore_wait(barrier_sem, 2)

        comm_ref[0] = partial
        out_ref[...] = partial.reshape(B, SQ, D)

        for h in range(N_DEV - 1):
            rdma = pltpu.make_async_remote_copy(
                src_ref=comm_ref.at[h],
                dst_ref=comm_ref.at[h + 1],
                send_sem=send_sems.at[h],
                recv_sem=recv_sems.at[h],
                device_id=(right,),
                device_id_type=pl.DeviceIdType.MESH,
            )
            rdma.start()
            rdma.wait()
            out_ref[...] = out_ref[...] + comm_ref[h + 1].reshape(B, SQ, D)

    return pl.pallas_call(
        body,
        out_shape=jax.ShapeDtypeStruct((B, SQ, D), jnp.float32),
        in_specs=[pl.BlockSpec(memory_space=pltpu.VMEM)] * 5,
        out_specs=pl.BlockSpec(memory_space=pltpu.VMEM),
        scratch_shapes=[
            pltpu.VMEM((N_DEV, BSQ, D), jnp.float32),
            pltpu.SemaphoreType.DMA((N_DEV - 1,)),
            pltpu.SemaphoreType.DMA((N_DEV - 1,)),
        ],
        compiler_params=pltpu.CompilerParams(collective_id=0),
    )(x, Wq, Wo, Wk, Wv)


# device time: 30628 ns/iter; 2.2349x vs baseline; 2.2349x over previous
import jax
import jax.numpy as jnp
from jax import lax
from jax.experimental import pallas as pl
from jax.experimental.pallas import tpu as pltpu

N_DEV = 8
B, SQ, D = 2, 128, 512
HQ_PER = 8
DH = 64
BSQ = B * SQ


def kernel(x, Wq, Wo, Wk, Wv):
    def body(x_ref, wq_ref, wo_ref, wk_ref, wv_ref, out_ref,
             send_ref, recv_ref, send_sems, recv_sems):
        my = lax.axis_index("i")

        x2 = x_ref[...].reshape(BSQ, D).astype(jnp.bfloat16)
        wq = wq_ref[...].astype(jnp.bfloat16)
        q = jnp.dot(x2, wq, preferred_element_type=jnp.float32)
        q = q.astype(jnp.bfloat16)

        kv_start = my * (2 * DH)
        wk = wk_ref[:, pl.ds(kv_start, 2 * DH)].astype(jnp.bfloat16)
        wv = wv_ref[:, pl.ds(kv_start, 2 * DH)].astype(jnp.bfloat16)
        k = jnp.dot(x2, wk, preferred_element_type=jnp.float32).astype(jnp.bfloat16)
        v = jnp.dot(x2, wv, preferred_element_type=jnp.float32).astype(jnp.bfloat16)

        ctx_rows = []
        for b in range(B):
            r0 = b * SQ
            head_ctx = []
            for h in range(HQ_PER):
                kvh = h // 4
                qh = q[r0:r0 + SQ, h * DH:(h + 1) * DH]
                kh = k[r0:r0 + SQ, kvh * DH:(kvh + 1) * DH]
                vh = v[r0:r0 + SQ, kvh * DH:(kvh + 1) * DH]
                s = jnp.dot(qh, kh.T, preferred_element_type=jnp.float32) * 0.125
                m = jnp.max(s, axis=-1, keepdims=True)
                p = jnp.exp(s - m)
                l = jnp.sum(p, axis=-1, keepdims=True)
                o = jnp.dot(p.astype(jnp.bfloat16), vh,
                            preferred_element_type=jnp.float32) / l
                head_ctx.append(o.astype(jnp.bfloat16))
            ctx_rows.append(jnp.concatenate(head_ctx, axis=1))
        ctx = jnp.concatenate(ctx_rows, axis=0)

        partial = jnp.dot(ctx, wo_ref[...].astype(jnp.bfloat16),
                          preferred_element_type=jnp.float32)

        partners = [jnp.bitwise_xor(my, 1 << r) for r in range(3)]

        barrier_sem = pltpu.get_barrier_semaphore()
        for p in partners:
            pl.semaphore_signal(barrier_sem, inc=1, device_id=(p,),
                                device_id_type=pl.DeviceIdType.MESH)
        pl.semaphore_wait(barrier_sem, 3)

        acc = partial
        for r in range(3):
            send_ref[r] = acc.astype(jnp.bfloat16)
            rdma = pltpu.make_async_remote_copy(
                src_ref=send_ref.at[r],
                dst_ref=recv_ref.at[r],
                send_sem=send_sems.at[r],
                recv_sem=recv_sems.at[r],
                device_id=(partners[r],),
                device_id_type=pl.DeviceIdType.MESH,
            )
            rdma.start()
            rdma.wait()
            acc = acc + recv_ref[r].astype(jnp.float32)
        out_ref[...] = acc.reshape(B, SQ, D)

    return pl.pallas_call(
        body,
        out_shape=jax.ShapeDtypeStruct((B, SQ, D), jnp.float32),
        in_specs=[pl.BlockSpec(memory_space=pltpu.VMEM)] * 5,
        out_specs=pl.BlockSpec(memory_space=pltpu.VMEM),
        scratch_shapes=[
            pltpu.VMEM((3, BSQ, D), jnp.bfloat16),
            pltpu.VMEM((3, BSQ, D), jnp.bfloat16),
            pltpu.SemaphoreType.DMA((3,)),
            pltpu.SemaphoreType.DMA((3,)),
        ],
        compiler_params=pltpu.CompilerParams(collective_id=0),
    )(x, Wq, Wo, Wk, Wv)


# device time: 28066 ns/iter; 2.4389x vs baseline; 1.0913x over previous
import jax
import jax.numpy as jnp
from jax import lax
from jax.experimental import pallas as pl
from jax.experimental.pallas import tpu as pltpu

N_DEV = 8
B, SQ, D = 2, 128, 512
HQ_PER = 8
DH = 64
BSQ = B * SQ


def kernel(x, Wq, Wo, Wk, Wv):
    def body(x_ref, wq_ref, wo_ref, wk_ref, wv_ref, out_ref,
             send_ref, recv_ref, send_sems, recv_sems):
        my = lax.axis_index("i")

        x2 = x_ref[...].reshape(BSQ, D).astype(jnp.bfloat16)
        wq = wq_ref[...].astype(jnp.bfloat16)
        q = jnp.dot(x2, wq, preferred_element_type=jnp.float32)
        q = q.astype(jnp.bfloat16)

        kv_start = my * (2 * DH)
        wk = wk_ref[:, pl.ds(kv_start, 2 * DH)].astype(jnp.bfloat16)
        wv = wv_ref[:, pl.ds(kv_start, 2 * DH)].astype(jnp.bfloat16)
        k = jnp.dot(x2, wk, preferred_element_type=jnp.float32).astype(jnp.bfloat16)
        v = jnp.dot(x2, wv, preferred_element_type=jnp.float32).astype(jnp.bfloat16)

        ctx_rows = []
        for b in range(B):
            r0 = b * SQ
            head_ctx = [None] * HQ_PER
            for kvh in range(2):
                qs = jnp.concatenate(
                    [q[r0:r0 + SQ, (4 * kvh + j) * DH:(4 * kvh + j + 1) * DH]
                     for j in range(4)], axis=0)
                kh = k[r0:r0 + SQ, kvh * DH:(kvh + 1) * DH]
                vh = v[r0:r0 + SQ, kvh * DH:(kvh + 1) * DH]
                s = jnp.dot(qs, kh.T, preferred_element_type=jnp.float32) * 0.125
                m = jnp.max(s, axis=-1, keepdims=True)
                p = jnp.exp(s - m)
                l = jnp.sum(p, axis=-1, keepdims=True)
                o = jnp.dot(p.astype(jnp.bfloat16), vh,
                            preferred_element_type=jnp.float32) / l
                o = o.astype(jnp.bfloat16)
                for j in range(4):
                    head_ctx[4 * kvh + j] = o[j * SQ:(j + 1) * SQ, :]
            ctx_rows.append(jnp.concatenate(head_ctx, axis=1))
        ctx = jnp.concatenate(ctx_rows, axis=0)

        partial = jnp.dot(ctx, wo_ref[...].astype(jnp.bfloat16),
                          preferred_element_type=jnp.float32)

        partners = [jnp.bitwise_xor(my, 1 << r) for r in range(3)]

        barrier_sem = pltpu.get_barrier_semaphore()
        for p in partners:
            pl.semaphore_signal(barrier_sem, inc=1, device_id=(p,),
                                device_id_type=pl.DeviceIdType.MESH)
        pl.semaphore_wait(barrier_sem, 3)

        acc = partial
        for r in range(3):
            send_ref[r] = acc.astype(jnp.bfloat16)
            rdma = pltpu.make_async_remote_copy(
                src_ref=send_ref.at[r],
                dst_ref=recv_ref.at[r],
                send_sem=send_sems.at[r],
                recv_sem=recv_sems.at[r],
                device_id=(partners[r],),
                device_id_type=pl.DeviceIdType.MESH,
            )
            rdma.start()
            rdma.wait()
            acc = acc + recv_ref[r].astype(jnp.float32)
        out_ref[...] = acc.reshape(B, SQ, D)

    return pl.pallas_call(
        body,
        out_shape=jax.ShapeDtypeStruct((B, SQ, D), jnp.float32),
        in_specs=[pl.BlockSpec(memory_space=pltpu.VMEM)] * 5,
        out_specs=pl.BlockSpec(memory_space=pltpu.VMEM),
        scratch_shapes=[
            pltpu.VMEM((3, BSQ, D), jnp.bfloat16),
            pltpu.VMEM((3, BSQ, D), jnp.bfloat16),
            pltpu.SemaphoreType.DMA((3,)),
            pltpu.SemaphoreType.DMA((3,)),
        ],
        compiler_params=pltpu.CompilerParams(collective_id=0),
    )(x, Wq, Wo, Wk, Wv)


# device time: 27075 ns/iter; 2.5282x vs baseline; 1.0366x over previous
import jax
import jax.numpy as jnp
from jax import lax
from jax.experimental import pallas as pl
from jax.experimental.pallas import tpu as pltpu

N_DEV = 8
B, SQ, D = 2, 128, 512
HQ_PER = 8
DH = 64
BSQ = B * SQ
MASKS = (1, 3, 4)


def kernel(x, Wq, Wo, Wk, Wv):
    def body(x_hbm, wq_hbm, wo_hbm, wk_hbm, wv_hbm, out_ref,
             x_v, wq_v, wo_v, wk_v, wv_v,
             load_sems, send_ref, recv_ref, send_sems, recv_sems):
        my = lax.axis_index("i")

        kv_start = my * (2 * DH)
        loads = [
            pltpu.make_async_copy(x_hbm, x_v, load_sems.at[0]),
            pltpu.make_async_copy(wq_hbm, wq_v, load_sems.at[1]),
            pltpu.make_async_copy(wk_hbm.at[:, pl.ds(kv_start, 2 * DH)],
                                  wk_v, load_sems.at[2]),
            pltpu.make_async_copy(wv_hbm.at[:, pl.ds(kv_start, 2 * DH)],
                                  wv_v, load_sems.at[3]),
            pltpu.make_async_copy(wo_hbm, wo_v, load_sems.at[4]),
        ]
        for ld in loads:
            ld.start()

        loads[0].wait()
        loads[1].wait()
        x2 = x_v[...].reshape(BSQ, D).astype(jnp.bfloat16)
        q = jnp.dot(x2, wq_v[...].astype(jnp.bfloat16),
                    preferred_element_type=jnp.float32)
        q = q.astype(jnp.bfloat16)

        loads[2].wait()
        loads[3].wait()
        k = jnp.dot(x2, wk_v[...].astype(jnp.bfloat16),
                    preferred_element_type=jnp.float32).astype(jnp.bfloat16)
        v = jnp.dot(x2, wv_v[...].astype(jnp.bfloat16),
                    preferred_element_type=jnp.float32).astype(jnp.bfloat16)

        ctx_rows = []
        for b in range(B):
            r0 = b * SQ
            head_ctx = [None] * HQ_PER
            for kvh in range(2):
                qs = jnp.concatenate(
                    [q[r0:r0 + SQ, (4 * kvh + j) * DH:(4 * kvh + j + 1) * DH]
                     for j in range(4)], axis=0)
                kh = k[r0:r0 + SQ, kvh * DH:(kvh + 1) * DH]
                vh = v[r0:r0 + SQ, kvh * DH:(kvh + 1) * DH]
                s = jnp.dot(qs, kh.T, preferred_element_type=jnp.float32) * 0.125
                m = jnp.max(s, axis=-1, keepdims=True)
                p = jnp.exp(s - m)
                l = jnp.sum(p, axis=-1, keepdims=True)
                o = jnp.dot(p.astype(jnp.bfloat16), vh,
                            preferred_element_type=jnp.float32) / l
                o = o.astype(jnp.bfloat16)
                for j in range(4):
                    head_ctx[4 * kvh + j] = o[j * SQ:(j + 1) * SQ, :]
            ctx_rows.append(jnp.concatenate(head_ctx, axis=1))
        ctx = jnp.concatenate(ctx_rows, axis=0)

        loads[4].wait()
        partial = jnp.dot(ctx, wo_v[...].astype(jnp.bfloat16),
                          preferred_element_type=jnp.float32)

        partners = [jnp.bitwise_xor(my, mk) for mk in MASKS]

        barrier_sem = pltpu.get_barrier_semaphore()
        for p in partners:
            pl.semaphore_signal(barrier_sem, inc=1, device_id=(p,),
                                device_id_type=pl.DeviceIdType.MESH)
        pl.semaphore_wait(barrier_sem, 3)

        acc = partial
        for r in range(3):
            send_ref[r] = acc.astype(jnp.bfloat16)
            rdma = pltpu.make_async_remote_copy(
                src_ref=send_ref.at[r],
                dst_ref=recv_ref.at[r],
                send_sem=send_sems.at[r],
                recv_sem=recv_sems.at[r],
                device_id=(partners[r],),
                device_id_type=pl.DeviceIdType.MESH,
            )
            rdma.start()
            rdma.wait()
            acc = acc + recv_ref[r].astype(jnp.float32)
        out_ref[...] = acc.reshape(B, SQ, D)

    return pl.pallas_call(
        body,
        out_shape=jax.ShapeDtypeStruct((B, SQ, D), jnp.float32),
        in_specs=[pl.BlockSpec(memory_space=pl.ANY)] * 5,
        out_specs=pl.BlockSpec(memory_space=pltpu.VMEM),
        scratch_shapes=[
            pltpu.VMEM((B, SQ, D), jnp.float32),
            pltpu.VMEM((D, HQ_PER * DH), jnp.float32),
            pltpu.VMEM((HQ_PER * DH, D), jnp.float32),
            pltpu.VMEM((D, 2 * DH), jnp.float32),
            pltpu.VMEM((D, 2 * DH), jnp.float32),
            pltpu.SemaphoreType.DMA((5,)),
            pltpu.VMEM((3, BSQ, D), jnp.bfloat16),
            pltpu.VMEM((3, BSQ, D), jnp.bfloat16),
            pltpu.SemaphoreType.DMA((3,)),
            pltpu.SemaphoreType.DMA((3,)),
        ],
        compiler_params=pltpu.CompilerParams(collective_id=0),
    )(x, Wq, Wo, Wk, Wv)


# device time: 27046 ns/iter; 2.5309x vs baseline; 1.0011x over previous
import jax
import jax.numpy as jnp
from jax import lax
from jax.experimental import pallas as pl
from jax.experimental.pallas import tpu as pltpu

N_DEV = 8
B, SQ, D = 2, 128, 512
HQ_PER = 8
DH = 64
BSQ = B * SQ
MASKS = (1, 3, 4)


def kernel(x, Wq, Wo, Wk, Wv):
    def body(x_hbm, wq_hbm, wo_hbm, wk_hbm, wv_hbm, out_ref,
             x_v, wq_v, wo_v, wk_v, wv_v, out_v,
             load_sems, send_ref, recv_ref, send_sems, recv_sems):
        my = lax.axis_index("i")

        kv_start = my * (2 * DH)
        loads = [
            pltpu.make_async_copy(x_hbm, x_v, load_sems.at[0]),
            pltpu.make_async_copy(wq_hbm, wq_v, load_sems.at[1]),
            pltpu.make_async_copy(wk_hbm.at[:, pl.ds(kv_start, 2 * DH)],
                                  wk_v, load_sems.at[2]),
            pltpu.make_async_copy(wv_hbm.at[:, pl.ds(kv_start, 2 * DH)],
                                  wv_v, load_sems.at[3]),
            pltpu.make_async_copy(wo_hbm, wo_v, load_sems.at[4]),
        ]
        for ld in loads:
            ld.start()

        loads[0].wait()
        loads[1].wait()
        x2 = x_v[...].reshape(BSQ, D).astype(jnp.bfloat16)
        q = jnp.dot(x2, wq_v[...].astype(jnp.bfloat16),
                    preferred_element_type=jnp.float32)
        q = q.astype(jnp.bfloat16)

        loads[2].wait()
        loads[3].wait()
        k = jnp.dot(x2, wk_v[...].astype(jnp.bfloat16),
                    preferred_element_type=jnp.float32).astype(jnp.bfloat16)
        v = jnp.dot(x2, wv_v[...].astype(jnp.bfloat16),
                    preferred_element_type=jnp.float32).astype(jnp.bfloat16)

        ctx_rows = []
        for b in range(B):
            r0 = b * SQ
            head_ctx = [None] * HQ_PER
            for kvh in range(2):
                qs = jnp.concatenate(
                    [q[r0:r0 + SQ, (4 * kvh + j) * DH:(4 * kvh + j + 1) * DH]
                     for j in range(4)], axis=0)
                kh = k[r0:r0 + SQ, kvh * DH:(kvh + 1) * DH]
                vh = v[r0:r0 + SQ, kvh * DH:(kvh + 1) * DH]
                s = jnp.dot(qs, kh.T, preferred_element_type=jnp.float32) * 0.125
                m = jnp.max(s, axis=-1, keepdims=True)
                p = jnp.exp(s - m)
                l = jnp.sum(p, axis=-1, keepdims=True)
                o = jnp.dot(p.astype(jnp.bfloat16), vh,
                            preferred_element_type=jnp.float32) / l
                o = o.astype(jnp.bfloat16)
                for j in range(4):
                    head_ctx[4 * kvh + j] = o[j * SQ:(j + 1) * SQ, :]
            ctx_rows.append(jnp.concatenate(head_ctx, axis=1))
        ctx = jnp.concatenate(ctx_rows, axis=0)

        loads[4].wait()
        partial = jnp.dot(ctx, wo_v[...].astype(jnp.bfloat16),
                          preferred_element_type=jnp.float32)

        partners = [jnp.bitwise_xor(my, mk) for mk in MASKS]

        barrier_sem = pltpu.get_barrier_semaphore()
        for p in partners:
            pl.semaphore_signal(barrier_sem, inc=1, device_id=(p,),
                                device_id_type=pl.DeviceIdType.MESH)
        pl.semaphore_wait(barrier_sem, 3)

        acc = partial
        for r in range(3):
            send_ref[r] = acc.astype(jnp.bfloat16)
            rdma = pltpu.make_async_remote_copy(
                src_ref=send_ref.at[r],
                dst_ref=recv_ref.at[r],
                send_sem=send_sems.at[r],
                recv_sem=recv_sems.at[r],
                device_id=(partners[r],),
                device_id_type=pl.DeviceIdType.MESH,
            )
            rdma.start()
            rdma.wait()
            acc = acc + recv_ref[r].astype(jnp.float32)
        out_v[...] = acc.reshape(B, SQ, D)
        out_copy = pltpu.make_async_copy(out_v, out_ref, load_sems.at[0])
        out_copy.start()
        out_copy.wait()

    return pl.pallas_call(
        body,
        out_shape=jax.ShapeDtypeStruct((B, SQ, D), jnp.float32),
        in_specs=[pl.BlockSpec(memory_space=pltpu.MemorySpace.HBM)] * 5,
        out_specs=pl.BlockSpec(memory_space=pltpu.MemorySpace.HBM),
        scratch_shapes=[
            pltpu.VMEM((B, SQ, D), jnp.float32),
            pltpu.VMEM((D, HQ_PER * DH), jnp.float32),
            pltpu.VMEM((HQ_PER * DH, D), jnp.float32),
            pltpu.VMEM((D, 2 * DH), jnp.float32),
            pltpu.VMEM((D, 2 * DH), jnp.float32),
            pltpu.VMEM((B, SQ, D), jnp.float32),
            pltpu.SemaphoreType.DMA((5,)),
            pltpu.VMEM((3, BSQ, D), jnp.bfloat16),
            pltpu.VMEM((3, BSQ, D), jnp.bfloat16),
            pltpu.SemaphoreType.DMA((3,)),
            pltpu.SemaphoreType.DMA((3,)),
        ],
        compiler_params=pltpu.CompilerParams(collective_id=0),
    )(x, Wq, Wo, Wk, Wv)


# device time: 21584 ns/iter; 3.1713x vs baseline; 1.2531x over previous
import jax
import jax.numpy as jnp
from jax import lax
from jax.experimental import pallas as pl
from jax.experimental.pallas import tpu as pltpu

N_DEV = 8
B, SQ, D = 2, 128, 512
HQ_PER = 8
DH = 64
BSQ = B * SQ
MASKS = (1, 3, 4)


def kernel(x, Wq, Wo, Wk, Wv):
    def body(x_hbm, wq_hbm, wo_hbm, wk_hbm, wv_hbm, out_ref,
             x_v, wq_v, wo_v, wk_v, wv_v, out_v,
             load_sems, send_ref, recv_ref, send_sems, recv_sems):
        my = lax.axis_index("i")

        kv_start = my * (2 * DH)
        loads = [
            pltpu.make_async_copy(x_hbm, x_v, load_sems.at[0]),
            pltpu.make_async_copy(wq_hbm, wq_v, load_sems.at[1]),
            pltpu.make_async_copy(wk_hbm.at[:, pl.ds(kv_start, 2 * DH)],
                                  wk_v, load_sems.at[2]),
            pltpu.make_async_copy(wv_hbm.at[:, pl.ds(kv_start, 2 * DH)],
                                  wv_v, load_sems.at[3]),
            pltpu.make_async_copy(wo_hbm, wo_v, load_sems.at[4]),
        ]
        for ld in loads:
            ld.start()

        loads[0].wait()
        loads[1].wait()
        x2 = x_v[...].reshape(BSQ, D).astype(jnp.bfloat16)
        q = jnp.dot(x2, wq_v[...].astype(jnp.bfloat16),
                    preferred_element_type=jnp.float32)
        q = q.astype(jnp.bfloat16)

        loads[2].wait()
        loads[3].wait()
        k = jnp.dot(x2, wk_v[...].astype(jnp.bfloat16),
                    preferred_element_type=jnp.float32).astype(jnp.bfloat16)
        v = jnp.dot(x2, wv_v[...].astype(jnp.bfloat16),
                    preferred_element_type=jnp.float32).astype(jnp.bfloat16)

        ctx_rows = []
        for b in range(B):
            r0 = b * SQ
            head_ctx = [None] * HQ_PER
            for kvh in range(2):
                qs = jnp.concatenate(
                    [q[r0:r0 + SQ, (4 * kvh + j) * DH:(4 * kvh + j + 1) * DH]
                     for j in range(4)], axis=0)
                kh = k[r0:r0 + SQ, kvh * DH:(kvh + 1) * DH]
                vh = v[r0:r0 + SQ, kvh * DH:(kvh + 1) * DH]
                s = jnp.dot(qs, kh.T, preferred_element_type=jnp.float32) * 0.125
                m = jnp.max(s, axis=-1, keepdims=True)
                p = jnp.exp(s - m)
                l = jnp.sum(p, axis=-1, keepdims=True)
                o = jnp.dot(p.astype(jnp.bfloat16), vh,
                            preferred_element_type=jnp.float32) / l
                o = o.astype(jnp.bfloat16)
                for j in range(4):
                    head_ctx[4 * kvh + j] = o[j * SQ:(j + 1) * SQ, :]
            ctx_rows.append(jnp.concatenate(head_ctx, axis=1))
        ctx = jnp.concatenate(ctx_rows, axis=0)

        loads[4].wait()
        partial = jnp.dot(ctx, wo_v[...].astype(jnp.bfloat16),
                          preferred_element_type=jnp.float32)

        partners = [jnp.bitwise_xor(my, mk) for mk in MASKS]

        barrier_sem = pltpu.get_barrier_semaphore()
        for p in partners:
            pl.semaphore_signal(barrier_sem, inc=1, device_id=(p,),
                                device_id_type=pl.DeviceIdType.MESH)
        pl.semaphore_wait(barrier_sem, 3)

        acc = partial
        for r in range(3):
            send_ref[r] = acc.astype(jnp.bfloat16)
            rdma = pltpu.make_async_remote_copy(
                src_ref=send_ref.at[r],
                dst_ref=recv_ref.at[r],
                send_sem=send_sems.at[r],
                recv_sem=recv_sems.at[r],
                device_id=(partners[r],),
                device_id_type=pl.DeviceIdType.MESH,
            )
            rdma.start()
            rdma.wait()
            acc = acc + recv_ref[r].astype(jnp.float32)
        out_v[...] = acc.reshape(B, SQ, D)
        out_copy = pltpu.make_async_copy(out_v, out_ref, load_sems.at[0])
        out_copy.start()
        out_copy.wait()

    return pl.pallas_call(
        body,
        out_shape=jax.ShapeDtypeStruct((B, SQ, D), jnp.float32),
        in_specs=[pl.BlockSpec(memory_space=pltpu.MemorySpace.HBM)] * 5,
        out_specs=pl.BlockSpec(memory_space=pltpu.MemorySpace.HBM),
        scratch_shapes=[
            pltpu.VMEM((B, SQ, D), jnp.float32),
            pltpu.VMEM((D, HQ_PER * DH), jnp.float32),
            pltpu.VMEM((HQ_PER * DH, D), jnp.float32),
            pltpu.VMEM((D, 2 * DH), jnp.float32),
            pltpu.VMEM((D, 2 * DH), jnp.float32),
            pltpu.VMEM((B, SQ, D), jnp.float32),
            pltpu.SemaphoreType.DMA((5,)),
            pltpu.VMEM((3, BSQ, D), jnp.bfloat16),
            pltpu.VMEM((3, BSQ, D), jnp.bfloat16),
            pltpu.SemaphoreType.DMA((3,)),
            pltpu.SemaphoreType.DMA((3,)),
        ],
        compiler_params=pltpu.CompilerParams(collective_id=0),
    )(*(pltpu.with_memory_space_constraint(a, pltpu.MemorySpace.HBM)
        for a in (x, Wq, Wo, Wk, Wv)))
